# baseline (speedup 1.0000x reference)
import functools

import jax
import jax.numpy as jnp
from jax.experimental import pallas as pl
from jax.experimental.pallas import tpu as pltpu

_CHUNK_ROWS = 4096


def _stream_kernel(n_chunks, chunk, x_hbm, wt_ref, o_hbm,
                   xbuf, obuf, in_sems, out_sems):
    def in_copy(i, slot):
        return pltpu.make_async_copy(
            x_hbm.at[pl.ds(i * chunk, chunk), :],
            xbuf.at[slot],
            in_sems.at[slot],
        )

    def out_copy(i, slot):
        return pltpu.make_async_copy(
            obuf.at[slot],
            o_hbm.at[pl.ds(i * chunk, chunk), :],
            out_sems.at[slot],
        )

    in_copy(0, 0).start()
    for i in range(n_chunks):
        slot = i % 2
        if i + 1 < n_chunks:
            in_copy(i + 1, 1 - slot).start()
        in_copy(i, slot).wait()
        if i >= 2:
            out_copy(i - 2, slot).wait()
        obuf[slot] = jnp.dot(xbuf[slot], wt_ref[...],
                             preferred_element_type=jnp.float32)
        out_copy(i, slot).start()
    for i in range(max(0, n_chunks - 2), n_chunks):
        out_copy(i, i % 2).wait()


def kernel(x, W):
    B, T, D = x.shape
    rows = B * T
    xf = x.reshape(rows, D)
    wt = W.T
    chunk = min(_CHUNK_ROWS, rows)
    n_chunks = rows // chunk
    y = pl.pallas_call(
        functools.partial(_stream_kernel, n_chunks, chunk),
        in_specs=[
            pl.BlockSpec(memory_space=pl.ANY),
            pl.BlockSpec(memory_space=pltpu.MemorySpace.VMEM),
        ],
        out_specs=pl.BlockSpec(memory_space=pl.ANY),
        out_shape=jax.ShapeDtypeStruct((rows, D), x.dtype),
        scratch_shapes=[
            pltpu.VMEM((2, chunk, D), jnp.float32),
            pltpu.VMEM((2, chunk, D), jnp.float32),
            pltpu.SemaphoreType.DMA((2,)),
            pltpu.SemaphoreType.DMA((2,)),
        ],
    )(xf, wt)
    return y.reshape(B, T, D)


# speedup vs baseline: 1.2193x; 1.2193x over previous
import functools

import jax
import jax.numpy as jnp
from jax.experimental import pallas as pl
from jax.experimental.pallas import tpu as pltpu

_N_CHUNKS = 8


def _stream_kernel(n_chunks, chunk, x_hbm, wt_ref, o_hbm,
                   xbuf, obuf, in_sems, out_sems):
    def in_copy(i):
        return pltpu.make_async_copy(
            x_hbm.at[pl.ds(i * chunk, chunk), :],
            xbuf.at[i],
            in_sems.at[i],
        )

    def out_copy(i):
        return pltpu.make_async_copy(
            obuf.at[i],
            o_hbm.at[pl.ds(i * chunk, chunk), :],
            out_sems.at[i],
        )

    for i in range(n_chunks):
        in_copy(i).start()
    for i in range(n_chunks):
        in_copy(i).wait()
        obuf[i] = jnp.dot(xbuf[i], wt_ref[...],
                          preferred_element_type=jnp.float32)
        out_copy(i).start()
    for i in range(n_chunks):
        out_copy(i).wait()


def kernel(x, W):
    B, T, D = x.shape
    rows = B * T
    xf = x.reshape(rows, D)
    wt = W.T
    n_chunks = _N_CHUNKS
    chunk = rows // n_chunks
    y = pl.pallas_call(
        functools.partial(_stream_kernel, n_chunks, chunk),
        in_specs=[
            pl.BlockSpec(memory_space=pl.ANY),
            pl.BlockSpec(memory_space=pltpu.MemorySpace.VMEM),
        ],
        out_specs=pl.BlockSpec(memory_space=pl.ANY),
        out_shape=jax.ShapeDtypeStruct((rows, D), x.dtype),
        scratch_shapes=[
            pltpu.VMEM((n_chunks, chunk, D), jnp.float32),
            pltpu.VMEM((n_chunks, chunk, D), jnp.float32),
            pltpu.SemaphoreType.DMA((n_chunks,)),
            pltpu.SemaphoreType.DMA((n_chunks,)),
        ],
    )(xf, wt)
    return y.reshape(B, T, D)


# speedup vs baseline: 1.2475x; 1.0232x over previous
import functools

import jax
import jax.numpy as jnp
from jax.experimental import pallas as pl
from jax.experimental.pallas import tpu as pltpu

_N_CHUNKS = 8


def _stream_kernel(n_chunks, chunk, x_hbm, wt_ref, o_hbm,
                   xbuf, obuf, in_sems, out_sems):
    def in_copy(i):
        return pltpu.make_async_copy(
            x_hbm.at[pl.ds(i * chunk, chunk), :],
            xbuf.at[i],
            in_sems.at[i],
        )

    def out_copy(i):
        return pltpu.make_async_copy(
            obuf.at[i],
            o_hbm.at[pl.ds(i * chunk, chunk), :],
            out_sems.at[i],
        )

    for i in range(n_chunks):
        in_copy(i).start()
    wtb = wt_ref[...].astype(jnp.bfloat16)
    for i in range(n_chunks):
        in_copy(i).wait()
        obuf[i] = jnp.dot(xbuf[i].astype(jnp.bfloat16), wtb,
                          preferred_element_type=jnp.float32)
        out_copy(i).start()
    for i in range(n_chunks):
        out_copy(i).wait()


def kernel(x, W):
    B, T, D = x.shape
    rows = B * T
    xf = x.reshape(rows, D)
    wt = W.T
    n_chunks = _N_CHUNKS
    chunk = rows // n_chunks
    y = pl.pallas_call(
        functools.partial(_stream_kernel, n_chunks, chunk),
        in_specs=[
            pl.BlockSpec(memory_space=pl.ANY),
            pl.BlockSpec(memory_space=pltpu.MemorySpace.VMEM),
        ],
        out_specs=pl.BlockSpec(memory_space=pl.ANY),
        out_shape=jax.ShapeDtypeStruct((rows, D), x.dtype),
        scratch_shapes=[
            pltpu.VMEM((n_chunks, chunk, D), jnp.float32),
            pltpu.VMEM((n_chunks, chunk, D), jnp.float32),
            pltpu.SemaphoreType.DMA((n_chunks,)),
            pltpu.SemaphoreType.DMA((n_chunks,)),
        ],
    )(xf, wt)
    return y.reshape(B, T, D)


# speedup vs baseline: 1.2841x; 1.0294x over previous
import jax
import jax.numpy as jnp
from jax.experimental import pallas as pl
from jax.experimental.pallas import tpu as pltpu

_BLOCK_ROWS = 16384


def _linear_kernel(x_ref, wt_ref, o_ref):
    o_ref[...] = jnp.dot(x_ref[...].astype(jnp.bfloat16),
                         wt_ref[...].astype(jnp.bfloat16),
                         preferred_element_type=jnp.float32)


def kernel(x, W):
    B, T, D = x.shape
    rows = B * T
    xf = x.reshape(rows, D)
    wt = W.T
    block = min(_BLOCK_ROWS, rows)
    grid = pl.cdiv(rows, block)
    y = pl.pallas_call(
        _linear_kernel,
        grid=(grid,),
        in_specs=[
            pl.BlockSpec((block, D), lambda i: (i, 0)),
            pl.BlockSpec((D, D), lambda i: (0, 0)),
        ],
        out_specs=pl.BlockSpec((block, D), lambda i: (i, 0)),
        out_shape=jax.ShapeDtypeStruct((rows, D), x.dtype),
        compiler_params=pltpu.CompilerParams(
            dimension_semantics=("arbitrary",),
        ),
    )(xf, wt)
    return y.reshape(B, T, D)
